# Initial kernel scaffold; baseline (speedup 1.0000x reference)
import jax
import jax.numpy as jnp
from jax.experimental import pallas as pl


def kernel(locations, box_cls, box_regression, centerness, image_sizes):
    raise NotImplementedError("write your pallas kernel here")


# speedup vs baseline: 1.0147x; 1.0147x over previous
import jax
import jax.numpy as jnp
from jax.experimental import pallas as pl
from jax.experimental.pallas import tpu as pltpu

_PRE_NMS_THRESH = 0.05
_PRE_NMS_TOP_N = 1000
_NMS_THRESH = 0.6
_FPN_POST_NMS_TOP_N = 100
_STRIDE = 8.0


def _score_body(cls_ref, cent_ref, out_ref):
    x = cls_ref[...]
    cent = cent_ref[...]
    sig = 1.0 / (1.0 + jnp.exp(-x))
    sigc = 1.0 / (1.0 + jnp.exp(-cent))
    C = x.shape[1]
    cidx = jax.lax.broadcasted_iota(jnp.int32, x.shape, 1)
    cand = (sig > _PRE_NMS_THRESH) & (cidx < C - 1)
    out_ref[...] = jnp.where(cand, sig * sigc, -1.0)


def _masked_scores(box_cls, centerness):
    N, C, H, W = box_cls.shape
    HW = H * W
    cls3 = box_cls.reshape(N, C, HW)
    cent3 = centerness.reshape(N, 1, HW)
    blk = 2048
    grid = (N, HW // blk)
    return pl.pallas_call(
        _score_body,
        grid=grid,
        in_specs=[
            pl.BlockSpec((1, C, blk), lambda n, j: (n, 0, j)),
            pl.BlockSpec((1, 1, blk), lambda n, j: (n, 0, j)),
        ],
        out_specs=pl.BlockSpec((1, C, blk), lambda n, j: (n, 0, j)),
        out_shape=jax.ShapeDtypeStruct((N, C, HW), jnp.float32),
    )(cls3, cent3)


def _nms_keep(boxes, scores, valid):
    M = boxes.shape[0]
    order = jnp.argsort(-scores)
    b = boxes[order]
    v = valid[order]
    area = (b[:, 2] - b[:, 0]) * (b[:, 3] - b[:, 1])
    lt = jnp.maximum(b[:, None, :2], b[None, :, :2])
    rb = jnp.minimum(b[:, None, 2:], b[None, :, 2:])
    wh = jnp.clip(rb - lt, 0.0, None)
    inter = wh[..., 0] * wh[..., 1]
    iou = inter / jnp.clip(area[:, None] + area[None, :] - inter, 1e-9, None)
    idx = jnp.arange(M)
    def body(i, keep):
        sup = (iou[i] > _NMS_THRESH) & (idx > i)
        return jnp.where(keep[i], keep & (~sup), keep)
    keep_sorted = jax.lax.fori_loop(0, M, body, v)
    return jnp.zeros((M,), dtype=jnp.bool_).at[order].set(keep_sorted)


def kernel(locations, box_cls, box_regression, centerness, image_sizes):
    N, C, H, W = box_cls.shape
    HW = H * W
    masked_chw = _masked_scores(box_cls, centerness)
    masked = jnp.transpose(masked_chw, (0, 2, 1))
    reg = jnp.transpose(box_regression, (0, 2, 3, 1)).reshape(N, HW, 4) * _STRIDE
    flat = masked.reshape(N, HW * C)
    topv, topi = jax.lax.top_k(flat, _PRE_NMS_TOP_N)
    loc_idx = topi // C
    labels = topi % C + 1
    valid = topv > 0.0
    per_loc = locations[loc_idx]
    per_reg = jnp.take_along_axis(reg, loc_idx[..., None], axis=1)
    x1 = per_loc[..., 0] - per_reg[..., 0]
    y1 = per_loc[..., 1] - per_reg[..., 1]
    x2 = per_loc[..., 0] + per_reg[..., 2]
    y2 = per_loc[..., 1] + per_reg[..., 3]
    h_img = image_sizes[:, 0].astype(jnp.float32)[:, None]
    w_img = image_sizes[:, 1].astype(jnp.float32)[:, None]
    x1 = jnp.clip(x1, 0.0, w_img - 1.0)
    y1 = jnp.clip(y1, 0.0, h_img - 1.0)
    x2 = jnp.clip(x2, 0.0, w_img - 1.0)
    y2 = jnp.clip(y2, 0.0, h_img - 1.0)
    boxes = jnp.stack([x1, y1, x2, y2], axis=-1)
    scores = jnp.where(valid, jnp.sqrt(jnp.where(valid, topv, 1.0)), 0.0)
    max_coord = jnp.max(boxes)
    offs = labels.astype(jnp.float32) * (max_coord + 1.0)
    nms_boxes = boxes + offs[..., None]
    keep = jax.vmap(_nms_keep)(nms_boxes, scores, valid)
    kept_scores = jnp.where(keep, scores, -1.0)
    fv, fi = jax.lax.top_k(kept_scores, _FPN_POST_NMS_TOP_N)
    out_valid = fv > 0.0
    out_boxes = jnp.take_along_axis(boxes, fi[..., None], axis=1)
    out_scores = jnp.take_along_axis(scores, fi, axis=1)
    out_labels = jnp.take_along_axis(labels, fi, axis=1)
    out_boxes = jnp.where(out_valid[..., None], out_boxes, 0.0)
    out_scores = jnp.where(out_valid, out_scores, 0.0)
    out_labels = jnp.where(out_valid, out_labels, 0)
    return out_boxes, out_scores, out_labels, out_valid
